# baseline (speedup 1.0000x reference)
import functools

import jax
import jax.numpy as jnp
from jax import lax
from jax.experimental import pallas as pl
from jax.experimental.pallas import tpu as pltpu, tpu_sc as plsc

N = 20000
G = 128
B = 4
NW = 32
CHUNK = 640
NPAD = NW * CHUNK
NV = CHUNK // 16
NLAST = N - (NW - 1) * CHUNK

_MESH = plsc.VectorSubcoreMesh(
    core_axis_name="c", subcore_axis_name="s", num_cores=2, num_subcores=16)

_EPS = float(jnp.finfo(jnp.float32).eps)
_LN2 = 0.6931471805599453
_SQRT2 = 1.4142135381698608


def _ds16(v):
    return pl.ds(v * 16, 16)


def _wid():
    return lax.axis_index("s") * 2 + lax.axis_index("c")


def _ln(x):
    bits = lax.bitcast_convert_type(x, jnp.int32)
    e = lax.shift_right_logical(bits, 23) - 127
    mbits = (bits & jnp.int32(0x007FFFFF)) | jnp.int32(0x3F800000)
    m = lax.bitcast_convert_type(mbits, jnp.float32)
    big = m > _SQRT2
    m = jnp.where(big, m * 0.5, m)
    e = jnp.where(big, e + 1, e)
    z = m - 1.0
    s = z / (2.0 + z)
    s2 = s * s
    poly = 1.0 / 3.0 + s2 * (1.0 / 5.0 + s2 * (1.0 / 7.0 + s2 * (1.0 / 9.0)))
    lnm = 2.0 * s + 2.0 * s * s2 * poly
    return e.astype(jnp.float32) * _LN2 + lnm


def _load_anchors(anchor_t, tail_t, bbox_f, anc, bbt, wid, base):
    @pl.when(wid < NW - 1)
    def _():
        pltpu.sync_copy(anchor_t.at[:, pl.ds(base, CHUNK)], anc)

    @pl.when(wid == NW - 1)
    def _():
        pltpu.sync_copy(tail_t, anc)

    pltpu.sync_copy(bbox_f, bbt)


@functools.partial(
    pl.kernel,
    out_type=[
        jax.ShapeDtypeStruct((NPAD,), jnp.float32),
        jax.ShapeDtypeStruct((NPAD,), jnp.int32),
        jax.ShapeDtypeStruct((NW, G), jnp.float32),
        jax.ShapeDtypeStruct((NW, G), jnp.int32),
    ],
    mesh=_MESH,
    compiler_params=pltpu.CompilerParams(needs_layout_passes=False),
    scratch_types=[
        pltpu.VMEM((4, CHUNK), jnp.float32),
        pltpu.VMEM((4 * G,), jnp.float32),
        pltpu.VMEM((G,), jnp.float32),
        pltpu.VMEM((CHUNK,), jnp.float32),
        pltpu.VMEM((G, 16), jnp.float32),
        pltpu.VMEM((G, 16), jnp.int32),
        pltpu.VMEM((16 * G,), jnp.float32),
        pltpu.VMEM((16 * G,), jnp.int32),
        pltpu.VMEM((CHUNK,), jnp.float32),
        pltpu.VMEM((CHUNK,), jnp.int32),
        pltpu.VMEM((G,), jnp.float32),
        pltpu.VMEM((G,), jnp.int32),
    ],
)
def _phase1(anchor_t, tail_t, bbox_f, rowmax_hbm, rowarg_hbm, colval_hbm,
            colidx_hbm,
            anc, bbt, areab, areaa, colv, coli, colv_t, coli_t,
            rmax_b, rarg_b, cval, cidx):
    wid = _wid()
    base = wid * CHUNK
    lane = lax.broadcasted_iota(jnp.int32, (16,), 0)
    _load_anchors(anchor_t, tail_t, bbox_f, anc, bbt, wid, base)

    for jv in range(G // 16):
        b0 = bbt[pl.ds(0 * G + jv * 16, 16)]
        b1 = bbt[pl.ds(1 * G + jv * 16, 16)]
        b2 = bbt[pl.ds(2 * G + jv * 16, 16)]
        b3 = bbt[pl.ds(3 * G + jv * 16, 16)]
        areab[_ds16(jv)] = (b2 - b0) * (b3 - b1)

    def init_row(v, _):
        sl = _ds16(v)
        areaa[sl] = (anc[2, sl] - anc[0, sl]) * (anc[3, sl] - anc[1, sl])
        rmax_b[sl] = jnp.full((16,), -1.0, jnp.float32)
        rarg_b[sl] = jnp.zeros((16,), jnp.int32)
        return 0
    lax.fori_loop(0, NV, init_row, 0)

    def gt_blk(jv, _):
        bx1v = bbt[pl.ds(0 * G + jv * 16, 16)]
        by1v = bbt[pl.ds(1 * G + jv * 16, 16)]
        bx2v = bbt[pl.ds(2 * G + jv * 16, 16)]
        by2v = bbt[pl.ds(3 * G + jv * 16, 16)]
        abv = areab[pl.ds(jv * 16, 16)]
        for sb in range(16 // B):
            b1 = [jnp.full((16,), bx1v[sb * B + t], jnp.float32)
                  for t in range(B)]
            b2 = [jnp.full((16,), by1v[sb * B + t], jnp.float32)
                  for t in range(B)]
            b3 = [jnp.full((16,), bx2v[sb * B + t], jnp.float32)
                  for t in range(B)]
            b4 = [jnp.full((16,), by2v[sb * B + t], jnp.float32)
                  for t in range(B)]
            ab = [jnp.full((16,), abv[sb * B + t], jnp.float32)
                  for t in range(B)]
            jc = [jnp.full((16,), jv * 16 + (sb * B + t), jnp.int32)
                  for t in range(B)]

            def v_body(v, carry):
                cvs, cis = carry
                sl = _ds16(v)
                ax1, ay1 = anc[0, sl], anc[1, sl]
                ax2, ay2 = anc[2, sl], anc[3, sl]
                aa = areaa[sl]
                idxv = base + v * 16 + lane
                ious = []
                ncv = []
                nci = []
                for t in range(B):
                    tlx = jnp.maximum(ax1, b1[t])
                    tly = jnp.maximum(ay1, b2[t])
                    brx = jnp.minimum(ax2, b3[t])
                    bry = jnp.minimum(ay2, b4[t])
                    w = jnp.maximum(brx - tlx, jnp.float32(0.0))
                    h = jnp.maximum(bry - tly, jnp.float32(0.0))
                    inter = w * h
                    iou = inter / ((aa + ab[t]) - inter)
                    mc = iou > cvs[t]
                    ncv.append(jnp.where(mc, iou, cvs[t]))
                    nci.append(jnp.where(mc, idxv, cis[t]))
                    ious.append(iou)
                vals = list(ious)
                idxs = list(jc)
                while len(vals) > 1:
                    nv2 = []
                    ni2 = []
                    for p in range(0, len(vals), 2):
                        mp = vals[p + 1] > vals[p]
                        nv2.append(jnp.where(mp, vals[p + 1], vals[p]))
                        ni2.append(jnp.where(mp, idxs[p + 1], idxs[p]))
                    vals, idxs = nv2, ni2
                vf, gf = vals[0], idxs[0]
                rm = rmax_b[sl]
                ra = rarg_b[sl]
                mr = vf > rm
                rmax_b[sl] = jnp.where(mr, vf, rm)
                rarg_b[sl] = jnp.where(mr, gf, ra)
                return tuple(ncv), tuple(nci)

            neg1 = jnp.full((16,), -1.0, jnp.float32)
            zi = jnp.zeros((16,), jnp.int32)
            cvs, cis = plsc.parallel_loop(
                0, NV, carry=((neg1,) * B, (zi,) * B), unroll=4)(v_body)
            for t in range(B):
                j = jv * 16 + sb * B + t
                colv[j, :] = cvs[t]
                coli[j, :] = cis[t]
        return 0
    lax.fori_loop(0, G // 16, gt_blk, 0)

    def tr_body(j, _):
        jf = lane * G + j
        plsc.store_scatter(colv_t, [jf], colv[j, :])
        plsc.store_scatter(coli_t, [jf], coli[j, :])
        return 0
    lax.fori_loop(0, G, tr_body, 0)

    for jv in range(G // 16):
        sl = _ds16(jv)
        best = jnp.full((16,), -1.0, jnp.float32)
        bidx = jnp.zeros((16,), jnp.int32)
        for l in range(16):
            v = colv_t[pl.ds(l * G + jv * 16, 16)]
            i = coli_t[pl.ds(l * G + jv * 16, 16)]
            m = (v > best) | ((v == best) & (i < bidx))
            best = jnp.where(m, v, best)
            bidx = jnp.where(m, i, bidx)
        cval[sl] = best
        cidx[sl] = bidx

    pltpu.sync_copy(rmax_b, rowmax_hbm.at[pl.ds(base, CHUNK)])
    pltpu.sync_copy(rarg_b, rowarg_hbm.at[pl.ds(base, CHUNK)])
    pltpu.sync_copy(cval, colval_hbm.at[wid])
    pltpu.sync_copy(cidx, colidx_hbm.at[wid])


@functools.partial(
    pl.kernel,
    out_type=[
        jax.ShapeDtypeStruct((4, NPAD), jnp.float32),
        jax.ShapeDtypeStruct((NPAD,), jnp.int32),
    ],
    mesh=_MESH,
    compiler_params=pltpu.CompilerParams(needs_layout_passes=False),
    scratch_types=[
        pltpu.VMEM((4, CHUNK), jnp.float32),
        pltpu.VMEM((4 * G,), jnp.float32),
        pltpu.VMEM((CHUNK,), jnp.float32),
        pltpu.VMEM((CHUNK,), jnp.int32),
        pltpu.VMEM((NW, G), jnp.float32),
        pltpu.VMEM((NW, G), jnp.int32),
        pltpu.VMEM((G,), jnp.int32),
        pltpu.VMEM((CHUNK,), jnp.int32),
        pltpu.VMEM((4, CHUNK), jnp.float32),
    ],
)
def _phase2(anchor_t, tail_t, bbox_f, rowmax_hbm, rowarg_hbm, colval_hbm,
            colidx_hbm,
            loc_hbm, label_hbm,
            anc, bbt, rmax_b, rarg_b, cval, cidx, gta, lab, locv):
    wid = _wid()
    base = wid * CHUNK
    lane = lax.broadcasted_iota(jnp.int32, (16,), 0)
    _load_anchors(anchor_t, tail_t, bbox_f, anc, bbt, wid, base)
    pltpu.sync_copy(rowmax_hbm.at[pl.ds(base, CHUNK)], rmax_b)
    pltpu.sync_copy(rowarg_hbm.at[pl.ds(base, CHUNK)], rarg_b)
    pltpu.sync_copy(colval_hbm, cval)
    pltpu.sync_copy(colidx_hbm, cidx)

    lane0 = lane == 0

    for jv in range(G // 16):
        sl = _ds16(jv)

        def m_body(s, carry):
            best, bidx = carry
            v = cval[s, sl]
            i = cidx[s, sl]
            m = (v > best) | ((v == best) & (i < bidx))
            return jnp.where(m, v, best), jnp.where(m, i, bidx)

        best0 = jnp.full((16,), -1.0, jnp.float32)
        bidx0 = jnp.zeros((16,), jnp.int32)
        best, bidx = lax.fori_loop(0, NW, m_body, (best0, bidx0))
        gta[sl] = bidx

    def lab_body(v, _):
        sl = _ds16(v)
        mx = rmax_b[sl]
        lv = jnp.where(mx < 0.3, jnp.int32(0),
                       jnp.where(mx >= 0.7, jnp.int32(1), jnp.int32(-1)))
        lab[sl] = lv
        return 0
    lax.fori_loop(0, NV, lab_body, 0)

    def fix_body(jv, _):
        gv = gta[pl.ds(jv * 16, 16)]
        for l in range(16):
            j = jv * 16 + l
            li = gv[l] - base
            inb = (li >= 0) & (li < CHUNK)
            m = lane0 & inb
            liv = jnp.full((16,), li, jnp.int32)
            plsc.store_scatter(rarg_b, [liv], jnp.full((16,), j, jnp.int32),
                               mask=m)
            plsc.store_scatter(lab, [liv], jnp.full((16,), 1, jnp.int32),
                               mask=m)
        return 0
    lax.fori_loop(0, G // 16, fix_body, 0)

    def loc_body(v, _):
        sl = _ds16(v)
        g = rarg_b[sl]
        bx1 = plsc.load_gather(bbt, [g])
        by1 = plsc.load_gather(bbt, [g + G])
        bx2 = plsc.load_gather(bbt, [g + 2 * G])
        by2 = plsc.load_gather(bbt, [g + 3 * G])
        ax1, ay1, ax2, ay2 = anc[0, sl], anc[1, sl], anc[2, sl], anc[3, sl]
        w = ax2 - ax1
        h = ay2 - ay1
        ctrx = ax1 + 0.5 * w
        ctry = ay1 + 0.5 * h
        bw = bx2 - bx1
        bh = by2 - by1
        bctrx = bx1 + 0.5 * bw
        bctry = by1 + 0.5 * bh
        wc = jnp.maximum(w, _EPS)
        hc = jnp.maximum(h, _EPS)
        locv[0, sl] = (bctrx - ctrx) / wc
        locv[1, sl] = (bctry - ctry) / hc
        locv[2, sl] = _ln(bw / wc)
        locv[3, sl] = _ln(bh / hc)
        return 0
    lax.fori_loop(0, NV, loc_body, 0)

    pltpu.sync_copy(locv, loc_hbm.at[:, pl.ds(base, CHUNK)])
    pltpu.sync_copy(lab, label_hbm.at[pl.ds(base, CHUNK)])


@jax.jit
def kernel(anchor, bbox):
    anchor_t = anchor.T
    tail_t = jnp.pad(anchor_t[:, (NW - 1) * CHUNK:],
                     ((0, 0), (0, CHUNK - NLAST)))
    bbox_f = bbox.T.reshape(-1)
    rowmax, rowarg, colval, colidx = _phase1(anchor_t, tail_t, bbox_f)
    loc_t, label = _phase2(anchor_t, tail_t, bbox_f, rowmax, rowarg,
                           colval, colidx)
    return loc_t[:, :N].T, label[:N], rowmax[:N]


# speedup vs baseline: 1.0748x; 1.0748x over previous
import functools

import jax
import jax.numpy as jnp
from jax import lax
from jax.experimental import pallas as pl
from jax.experimental.pallas import tpu as pltpu, tpu_sc as plsc

N = 20000
G = 128
B = 4
NW = 32
CHUNK = 640
NPAD = NW * CHUNK
NV = CHUNK // 16
NLAST = N - (NW - 1) * CHUNK

_MESH = plsc.VectorSubcoreMesh(
    core_axis_name="c", subcore_axis_name="s", num_cores=2, num_subcores=16)

_EPS = float(jnp.finfo(jnp.float32).eps)
_LN2 = 0.6931471805599453
_SQRT2 = 1.4142135381698608


def _ds16(v):
    return pl.ds(v * 16, 16)


def _wid():
    return lax.axis_index("s") * 2 + lax.axis_index("c")


def _ln(x):
    bits = lax.bitcast_convert_type(x, jnp.int32)
    e = lax.shift_right_logical(bits, 23) - 127
    mbits = (bits & jnp.int32(0x007FFFFF)) | jnp.int32(0x3F800000)
    m = lax.bitcast_convert_type(mbits, jnp.float32)
    big = m > _SQRT2
    m = jnp.where(big, m * 0.5, m)
    e = jnp.where(big, e + 1, e)
    z = m - 1.0
    s = z / (2.0 + z)
    s2 = s * s
    poly = 1.0 / 3.0 + s2 * (1.0 / 5.0 + s2 * (1.0 / 7.0 + s2 * (1.0 / 9.0)))
    lnm = 2.0 * s + 2.0 * s * s2 * poly
    return e.astype(jnp.float32) * _LN2 + lnm


def _load_anchors(anchor_t, tail_t, bbox_f, anc, bbt, wid, base):
    @pl.when(wid < NW - 1)
    def _():
        pltpu.sync_copy(anchor_t.at[:, pl.ds(base, CHUNK)], anc)

    @pl.when(wid == NW - 1)
    def _():
        pltpu.sync_copy(tail_t, anc)

    pltpu.sync_copy(bbox_f, bbt)


@functools.partial(
    pl.kernel,
    out_type=[
        jax.ShapeDtypeStruct((NPAD,), jnp.float32),
        jax.ShapeDtypeStruct((NPAD,), jnp.int32),
        jax.ShapeDtypeStruct((NW, G), jnp.float32),
        jax.ShapeDtypeStruct((NW, G), jnp.int32),
    ],
    mesh=_MESH,
    compiler_params=pltpu.CompilerParams(needs_layout_passes=False),
    scratch_types=[
        pltpu.VMEM((4, CHUNK), jnp.float32),
        pltpu.VMEM((4 * G,), jnp.float32),
        pltpu.VMEM((G,), jnp.float32),
        pltpu.VMEM((CHUNK,), jnp.float32),
        pltpu.VMEM((G, 16), jnp.float32),
        pltpu.VMEM((G, 16), jnp.int32),
        pltpu.VMEM((16 * G,), jnp.float32),
        pltpu.VMEM((16 * G,), jnp.int32),
        pltpu.VMEM((CHUNK,), jnp.float32),
        pltpu.VMEM((CHUNK,), jnp.int32),
        pltpu.VMEM((G,), jnp.float32),
        pltpu.VMEM((G,), jnp.int32),
    ],
)
def _phase1(anchor_t, tail_t, bbox_f, rowmax_hbm, rowarg_hbm, colval_hbm,
            colidx_hbm,
            anc, bbt, areab, areaa, colv, coli, colv_t, coli_t,
            rmax_b, rarg_b, cval, cidx):
    wid = _wid()
    base = wid * CHUNK
    lane = lax.broadcasted_iota(jnp.int32, (16,), 0)
    _load_anchors(anchor_t, tail_t, bbox_f, anc, bbt, wid, base)

    for jv in range(G // 16):
        b0 = bbt[pl.ds(0 * G + jv * 16, 16)]
        b1 = bbt[pl.ds(1 * G + jv * 16, 16)]
        b2 = bbt[pl.ds(2 * G + jv * 16, 16)]
        b3 = bbt[pl.ds(3 * G + jv * 16, 16)]
        areab[_ds16(jv)] = (b2 - b0) * (b3 - b1)

    @plsc.parallel_loop(0, NV, unroll=2)
    def _(v):
        sl = _ds16(v)
        areaa[sl] = (anc[2, sl] - anc[0, sl]) * (anc[3, sl] - anc[1, sl])
        rmax_b[sl] = jnp.full((16,), -1.0, jnp.float32)
        rarg_b[sl] = jnp.zeros((16,), jnp.int32)

    def gt_blk(jv, _):
        bx1v = bbt[pl.ds(0 * G + jv * 16, 16)]
        by1v = bbt[pl.ds(1 * G + jv * 16, 16)]
        bx2v = bbt[pl.ds(2 * G + jv * 16, 16)]
        by2v = bbt[pl.ds(3 * G + jv * 16, 16)]
        abv = areab[pl.ds(jv * 16, 16)]
        for sb in range(16 // B):
            b1 = [jnp.full((16,), bx1v[sb * B + t], jnp.float32)
                  for t in range(B)]
            b2 = [jnp.full((16,), by1v[sb * B + t], jnp.float32)
                  for t in range(B)]
            b3 = [jnp.full((16,), bx2v[sb * B + t], jnp.float32)
                  for t in range(B)]
            b4 = [jnp.full((16,), by2v[sb * B + t], jnp.float32)
                  for t in range(B)]
            ab = [jnp.full((16,), abv[sb * B + t], jnp.float32)
                  for t in range(B)]
            jc = [jnp.full((16,), jv * 16 + (sb * B + t), jnp.int32)
                  for t in range(B)]

            def v_body(v, carry):
                cvs, cis = carry
                sl = _ds16(v)
                ax1, ay1 = anc[0, sl], anc[1, sl]
                ax2, ay2 = anc[2, sl], anc[3, sl]
                aa = areaa[sl]
                idxv = base + v * 16 + lane
                ious = []
                ncv = []
                nci = []
                for t in range(B):
                    tlx = jnp.maximum(ax1, b1[t])
                    tly = jnp.maximum(ay1, b2[t])
                    brx = jnp.minimum(ax2, b3[t])
                    bry = jnp.minimum(ay2, b4[t])
                    w = jnp.maximum(brx - tlx, jnp.float32(0.0))
                    h = jnp.maximum(bry - tly, jnp.float32(0.0))
                    inter = w * h
                    iou = inter / ((aa + ab[t]) - inter)
                    mc = iou > cvs[t]
                    ncv.append(jnp.where(mc, iou, cvs[t]))
                    nci.append(jnp.where(mc, idxv, cis[t]))
                    ious.append(iou)
                vals = list(ious)
                idxs = list(jc)
                while len(vals) > 1:
                    nv2 = []
                    ni2 = []
                    for p in range(0, len(vals), 2):
                        mp = vals[p + 1] > vals[p]
                        nv2.append(jnp.where(mp, vals[p + 1], vals[p]))
                        ni2.append(jnp.where(mp, idxs[p + 1], idxs[p]))
                    vals, idxs = nv2, ni2
                vf, gf = vals[0], idxs[0]
                rm = rmax_b[sl]
                ra = rarg_b[sl]
                mr = vf > rm
                rmax_b[sl] = jnp.where(mr, vf, rm)
                rarg_b[sl] = jnp.where(mr, gf, ra)
                return tuple(ncv), tuple(nci)

            neg1 = jnp.full((16,), -1.0, jnp.float32)
            zi = jnp.zeros((16,), jnp.int32)
            cvs, cis = plsc.parallel_loop(
                0, NV, carry=((neg1,) * B, (zi,) * B), unroll=2)(v_body)
            for t in range(B):
                j = jv * 16 + sb * B + t
                colv[j, :] = cvs[t]
                coli[j, :] = cis[t]
        return 0
    lax.fori_loop(0, G // 16, gt_blk, 0)

    @plsc.parallel_loop(0, G, unroll=2)
    def _(j):
        jf = lane * G + j
        plsc.store_scatter(colv_t, [jf], colv[j, :])
        plsc.store_scatter(coli_t, [jf], coli[j, :])

    for jv in range(G // 16):
        sl = _ds16(jv)
        best = jnp.full((16,), -1.0, jnp.float32)
        bidx = jnp.zeros((16,), jnp.int32)
        for l in range(16):
            v = colv_t[pl.ds(l * G + jv * 16, 16)]
            i = coli_t[pl.ds(l * G + jv * 16, 16)]
            m = (v > best) | ((v == best) & (i < bidx))
            best = jnp.where(m, v, best)
            bidx = jnp.where(m, i, bidx)
        cval[sl] = best
        cidx[sl] = bidx

    pltpu.sync_copy(rmax_b, rowmax_hbm.at[pl.ds(base, CHUNK)])
    pltpu.sync_copy(rarg_b, rowarg_hbm.at[pl.ds(base, CHUNK)])
    pltpu.sync_copy(cval, colval_hbm.at[wid])
    pltpu.sync_copy(cidx, colidx_hbm.at[wid])


@functools.partial(
    pl.kernel,
    out_type=[
        jax.ShapeDtypeStruct((4, NPAD), jnp.float32),
        jax.ShapeDtypeStruct((NPAD,), jnp.int32),
    ],
    mesh=_MESH,
    compiler_params=pltpu.CompilerParams(needs_layout_passes=False),
    scratch_types=[
        pltpu.VMEM((4, CHUNK), jnp.float32),
        pltpu.VMEM((4 * G,), jnp.float32),
        pltpu.VMEM((CHUNK,), jnp.float32),
        pltpu.VMEM((CHUNK,), jnp.int32),
        pltpu.VMEM((NW, G), jnp.float32),
        pltpu.VMEM((NW, G), jnp.int32),
        pltpu.VMEM((G,), jnp.int32),
        pltpu.VMEM((CHUNK,), jnp.int32),
        pltpu.VMEM((4, CHUNK), jnp.float32),
    ],
)
def _phase2(anchor_t, tail_t, bbox_f, rowmax_hbm, rowarg_hbm, colval_hbm,
            colidx_hbm,
            loc_hbm, label_hbm,
            anc, bbt, rmax_b, rarg_b, cval, cidx, gta, lab, locv):
    wid = _wid()
    base = wid * CHUNK
    lane = lax.broadcasted_iota(jnp.int32, (16,), 0)
    _load_anchors(anchor_t, tail_t, bbox_f, anc, bbt, wid, base)
    pltpu.sync_copy(rowmax_hbm.at[pl.ds(base, CHUNK)], rmax_b)
    pltpu.sync_copy(rowarg_hbm.at[pl.ds(base, CHUNK)], rarg_b)
    pltpu.sync_copy(colval_hbm, cval)
    pltpu.sync_copy(colidx_hbm, cidx)

    lane0 = lane == 0

    for jv in range(G // 16):
        sl = _ds16(jv)

        def m_body(s, carry):
            best, bidx = carry
            v = cval[s, sl]
            i = cidx[s, sl]
            m = (v > best) | ((v == best) & (i < bidx))
            return jnp.where(m, v, best), jnp.where(m, i, bidx)

        best0 = jnp.full((16,), -1.0, jnp.float32)
        bidx0 = jnp.zeros((16,), jnp.int32)
        best, bidx = lax.fori_loop(0, NW, m_body, (best0, bidx0))
        gta[sl] = bidx

    @plsc.parallel_loop(0, NV, unroll=2)
    def _(v):
        sl = _ds16(v)
        mx = rmax_b[sl]
        lab[sl] = jnp.where(mx < 0.3, jnp.int32(0),
                            jnp.where(mx >= 0.7, jnp.int32(1), jnp.int32(-1)))

    def fix_body(jv, _):
        gv = gta[pl.ds(jv * 16, 16)]
        for l in range(16):
            j = jv * 16 + l
            li = gv[l] - base
            inb = (li >= 0) & (li < CHUNK)
            m = lane0 & inb
            liv = jnp.full((16,), li, jnp.int32)
            plsc.store_scatter(rarg_b, [liv], jnp.full((16,), j, jnp.int32),
                               mask=m)
            plsc.store_scatter(lab, [liv], jnp.full((16,), 1, jnp.int32),
                               mask=m)
        return 0
    lax.fori_loop(0, G // 16, fix_body, 0)

    @plsc.parallel_loop(0, NV, unroll=2)
    def _(v):
        sl = _ds16(v)
        g = rarg_b[sl]
        bx1 = plsc.load_gather(bbt, [g])
        by1 = plsc.load_gather(bbt, [g + G])
        bx2 = plsc.load_gather(bbt, [g + 2 * G])
        by2 = plsc.load_gather(bbt, [g + 3 * G])
        ax1, ay1, ax2, ay2 = anc[0, sl], anc[1, sl], anc[2, sl], anc[3, sl]
        w = ax2 - ax1
        h = ay2 - ay1
        ctrx = ax1 + 0.5 * w
        ctry = ay1 + 0.5 * h
        bw = bx2 - bx1
        bh = by2 - by1
        bctrx = bx1 + 0.5 * bw
        bctry = by1 + 0.5 * bh
        wc = jnp.maximum(w, _EPS)
        hc = jnp.maximum(h, _EPS)
        locv[0, sl] = (bctrx - ctrx) / wc
        locv[1, sl] = (bctry - ctry) / hc
        locv[2, sl] = _ln(bw / wc)
        locv[3, sl] = _ln(bh / hc)

    pltpu.sync_copy(locv, loc_hbm.at[:, pl.ds(base, CHUNK)])
    pltpu.sync_copy(lab, label_hbm.at[pl.ds(base, CHUNK)])


@jax.jit
def kernel(anchor, bbox):
    anchor_t = anchor.T
    tail_t = jnp.pad(anchor_t[:, (NW - 1) * CHUNK:],
                     ((0, 0), (0, CHUNK - NLAST)))
    bbox_f = bbox.T.reshape(-1)
    rowmax, rowarg, colval, colidx = _phase1(anchor_t, tail_t, bbox_f)
    loc_t, label = _phase2(anchor_t, tail_t, bbox_f, rowmax, rowarg,
                           colval, colidx)
    return loc_t[:, :N].T, label[:N], rowmax[:N]


# speedup vs baseline: 1.1744x; 1.0927x over previous
import functools

import jax
import jax.numpy as jnp
from jax import lax
from jax.experimental import pallas as pl
from jax.experimental.pallas import tpu as pltpu, tpu_sc as plsc

N = 20000
G = 128
B = 4
NW = 32
CHUNK = 640
NPAD = NW * CHUNK
NV = CHUNK // 16
NLAST = N - (NW - 1) * CHUNK

_MESH = plsc.VectorSubcoreMesh(
    core_axis_name="c", subcore_axis_name="s", num_cores=2, num_subcores=16)

_EPS = float(jnp.finfo(jnp.float32).eps)
_LN2 = 0.6931471805599453
_SQRT2 = 1.4142135381698608


def _ds16(v):
    return pl.ds(v * 16, 16)


def _wid():
    return lax.axis_index("s") * 2 + lax.axis_index("c")


def _ln(x):
    bits = lax.bitcast_convert_type(x, jnp.int32)
    e = lax.shift_right_logical(bits, 23) - 127
    mbits = (bits & jnp.int32(0x007FFFFF)) | jnp.int32(0x3F800000)
    m = lax.bitcast_convert_type(mbits, jnp.float32)
    big = m > _SQRT2
    m = jnp.where(big, m * 0.5, m)
    e = jnp.where(big, e + 1, e)
    z = m - 1.0
    s = z / (2.0 + z)
    s2 = s * s
    poly = 1.0 / 3.0 + s2 * (1.0 / 5.0 + s2 * (1.0 / 7.0 + s2 * (1.0 / 9.0)))
    lnm = 2.0 * s + 2.0 * s * s2 * poly
    return e.astype(jnp.float32) * _LN2 + lnm


def _load_anchors(anchor_t, tail_t, bbox_f, anc, bbt, wid, base):
    @pl.when(wid < NW - 1)
    def _():
        pltpu.sync_copy(anchor_t.at[:, pl.ds(base, CHUNK)], anc)

    @pl.when(wid == NW - 1)
    def _():
        pltpu.sync_copy(tail_t, anc)

    pltpu.sync_copy(bbox_f, bbt)


@functools.partial(
    pl.kernel,
    out_type=[
        jax.ShapeDtypeStruct((NPAD,), jnp.float32),
        jax.ShapeDtypeStruct((NPAD,), jnp.int32),
        jax.ShapeDtypeStruct((NW, G), jnp.float32),
        jax.ShapeDtypeStruct((NW, G), jnp.int32),
    ],
    mesh=_MESH,
    compiler_params=pltpu.CompilerParams(needs_layout_passes=False),
    scratch_types=[
        pltpu.VMEM((4, CHUNK), jnp.float32),
        pltpu.VMEM((4 * G,), jnp.float32),
        pltpu.VMEM((G,), jnp.float32),
        pltpu.VMEM((CHUNK,), jnp.float32),
        pltpu.VMEM((G, 16), jnp.float32),
        pltpu.VMEM((G, 16), jnp.int32),
        pltpu.VMEM((16 * G,), jnp.float32),
        pltpu.VMEM((16 * G,), jnp.int32),
        pltpu.VMEM((CHUNK,), jnp.float32),
        pltpu.VMEM((CHUNK,), jnp.int32),
        pltpu.VMEM((G,), jnp.float32),
        pltpu.VMEM((G,), jnp.int32),
    ],
)
def _phase1(anchor_t, tail_t, bbox_f, rowmax_hbm, rowarg_hbm, colval_hbm,
            colidx_hbm,
            anc, bbt, areab, areaa, colv, coli, colv_t, coli_t,
            rmax_b, rarg_b, cval, cidx):
    wid = _wid()
    base = wid * CHUNK
    lane = lax.broadcasted_iota(jnp.int32, (16,), 0)
    _load_anchors(anchor_t, tail_t, bbox_f, anc, bbt, wid, base)

    for jv in range(G // 16):
        b0 = bbt[pl.ds(0 * G + jv * 16, 16)]
        b1 = bbt[pl.ds(1 * G + jv * 16, 16)]
        b2 = bbt[pl.ds(2 * G + jv * 16, 16)]
        b3 = bbt[pl.ds(3 * G + jv * 16, 16)]
        areab[_ds16(jv)] = (b2 - b0) * (b3 - b1)

    @plsc.parallel_loop(0, NV, unroll=2)
    def _(v):
        sl = _ds16(v)
        areaa[sl] = (anc[2, sl] - anc[0, sl]) * (anc[3, sl] - anc[1, sl])
        rmax_b[sl] = jnp.full((16,), -1.0, jnp.float32)
        rarg_b[sl] = jnp.zeros((16,), jnp.int32)

    def gt_blk(jv, _):
        bx1v = bbt[pl.ds(0 * G + jv * 16, 16)]
        by1v = bbt[pl.ds(1 * G + jv * 16, 16)]
        bx2v = bbt[pl.ds(2 * G + jv * 16, 16)]
        by2v = bbt[pl.ds(3 * G + jv * 16, 16)]
        abv = areab[pl.ds(jv * 16, 16)]
        for sb in range(16 // B):
            b1 = [jnp.full((16,), bx1v[sb * B + t], jnp.float32)
                  for t in range(B)]
            b2 = [jnp.full((16,), by1v[sb * B + t], jnp.float32)
                  for t in range(B)]
            b3 = [jnp.full((16,), bx2v[sb * B + t], jnp.float32)
                  for t in range(B)]
            b4 = [jnp.full((16,), by2v[sb * B + t], jnp.float32)
                  for t in range(B)]
            ab = [jnp.full((16,), abv[sb * B + t], jnp.float32)
                  for t in range(B)]
            jc = [jnp.full((16,), jv * 16 + (sb * B + t), jnp.int32)
                  for t in range(B)]

            def v_body(v, carry):
                cvs, cis = carry
                sl = _ds16(v)
                ax1, ay1 = anc[0, sl], anc[1, sl]
                ax2, ay2 = anc[2, sl], anc[3, sl]
                aa = areaa[sl]
                idxv = base + v * 16 + lane
                ious = []
                ncv = []
                nci = []
                for t in range(B):
                    tlx = jnp.maximum(ax1, b1[t])
                    tly = jnp.maximum(ay1, b2[t])
                    brx = jnp.minimum(ax2, b3[t])
                    bry = jnp.minimum(ay2, b4[t])
                    w = jnp.maximum(brx - tlx, jnp.float32(0.0))
                    h = jnp.maximum(bry - tly, jnp.float32(0.0))
                    inter = w * h
                    iou = inter / ((aa + ab[t]) - inter)
                    mc = iou > cvs[t]
                    ncv.append(jnp.where(mc, iou, cvs[t]))
                    nci.append(jnp.where(mc, idxv, cis[t]))
                    ious.append(iou)
                vals = list(ious)
                idxs = list(jc)
                while len(vals) > 1:
                    nv2 = []
                    ni2 = []
                    for p in range(0, len(vals), 2):
                        mp = vals[p + 1] > vals[p]
                        nv2.append(jnp.where(mp, vals[p + 1], vals[p]))
                        ni2.append(jnp.where(mp, idxs[p + 1], idxs[p]))
                    vals, idxs = nv2, ni2
                vf, gf = vals[0], idxs[0]
                rm = rmax_b[sl]
                ra = rarg_b[sl]
                mr = vf > rm
                rmax_b[sl] = jnp.where(mr, vf, rm)
                rarg_b[sl] = jnp.where(mr, gf, ra)
                return tuple(ncv), tuple(nci)

            neg1 = jnp.full((16,), -1.0, jnp.float32)
            zi = jnp.zeros((16,), jnp.int32)
            cvs, cis = plsc.parallel_loop(
                0, NV, carry=((neg1,) * B, (zi,) * B), unroll=1)(v_body)
            for t in range(B):
                j = jv * 16 + sb * B + t
                colv[j, :] = cvs[t]
                coli[j, :] = cis[t]
        return 0
    lax.fori_loop(0, G // 16, gt_blk, 0)

    @plsc.parallel_loop(0, G, unroll=2)
    def _(j):
        jf = lane * G + j
        plsc.store_scatter(colv_t, [jf], colv[j, :])
        plsc.store_scatter(coli_t, [jf], coli[j, :])

    for jv in range(G // 16):
        sl = _ds16(jv)
        best = jnp.full((16,), -1.0, jnp.float32)
        bidx = jnp.zeros((16,), jnp.int32)
        for l in range(16):
            v = colv_t[pl.ds(l * G + jv * 16, 16)]
            i = coli_t[pl.ds(l * G + jv * 16, 16)]
            m = (v > best) | ((v == best) & (i < bidx))
            best = jnp.where(m, v, best)
            bidx = jnp.where(m, i, bidx)
        cval[sl] = best
        cidx[sl] = bidx

    pltpu.sync_copy(rmax_b, rowmax_hbm.at[pl.ds(base, CHUNK)])
    pltpu.sync_copy(rarg_b, rowarg_hbm.at[pl.ds(base, CHUNK)])
    pltpu.sync_copy(cval, colval_hbm.at[wid])
    pltpu.sync_copy(cidx, colidx_hbm.at[wid])


@functools.partial(
    pl.kernel,
    out_type=[
        jax.ShapeDtypeStruct((4, NPAD), jnp.float32),
        jax.ShapeDtypeStruct((NPAD,), jnp.int32),
    ],
    mesh=_MESH,
    compiler_params=pltpu.CompilerParams(needs_layout_passes=False),
    scratch_types=[
        pltpu.VMEM((4, CHUNK), jnp.float32),
        pltpu.VMEM((4 * G,), jnp.float32),
        pltpu.VMEM((CHUNK,), jnp.float32),
        pltpu.VMEM((CHUNK,), jnp.int32),
        pltpu.VMEM((NW, G), jnp.float32),
        pltpu.VMEM((NW, G), jnp.int32),
        pltpu.VMEM((G,), jnp.int32),
        pltpu.VMEM((CHUNK,), jnp.int32),
        pltpu.VMEM((4, CHUNK), jnp.float32),
    ],
)
def _phase2(anchor_t, tail_t, bbox_f, rowmax_hbm, rowarg_hbm, colval_hbm,
            colidx_hbm,
            loc_hbm, label_hbm,
            anc, bbt, rmax_b, rarg_b, cval, cidx, gta, lab, locv):
    wid = _wid()
    base = wid * CHUNK
    lane = lax.broadcasted_iota(jnp.int32, (16,), 0)
    _load_anchors(anchor_t, tail_t, bbox_f, anc, bbt, wid, base)
    pltpu.sync_copy(rowmax_hbm.at[pl.ds(base, CHUNK)], rmax_b)
    pltpu.sync_copy(rowarg_hbm.at[pl.ds(base, CHUNK)], rarg_b)
    pltpu.sync_copy(colval_hbm, cval)
    pltpu.sync_copy(colidx_hbm, cidx)

    lane0 = lane == 0

    for jv in range(G // 16):
        sl = _ds16(jv)

        def m_body(s, carry):
            best, bidx = carry
            v = cval[s, sl]
            i = cidx[s, sl]
            m = (v > best) | ((v == best) & (i < bidx))
            return jnp.where(m, v, best), jnp.where(m, i, bidx)

        best0 = jnp.full((16,), -1.0, jnp.float32)
        bidx0 = jnp.zeros((16,), jnp.int32)
        best, bidx = lax.fori_loop(0, NW, m_body, (best0, bidx0))
        gta[sl] = bidx

    @plsc.parallel_loop(0, NV, unroll=2)
    def _(v):
        sl = _ds16(v)
        mx = rmax_b[sl]
        lab[sl] = jnp.where(mx < 0.3, jnp.int32(0),
                            jnp.where(mx >= 0.7, jnp.int32(1), jnp.int32(-1)))

    def fix_body(jv, _):
        gv = gta[pl.ds(jv * 16, 16)]
        for l in range(16):
            j = jv * 16 + l
            li = gv[l] - base
            inb = (li >= 0) & (li < CHUNK)
            m = lane0 & inb
            liv = jnp.full((16,), li, jnp.int32)
            plsc.store_scatter(rarg_b, [liv], jnp.full((16,), j, jnp.int32),
                               mask=m)
            plsc.store_scatter(lab, [liv], jnp.full((16,), 1, jnp.int32),
                               mask=m)
        return 0
    lax.fori_loop(0, G // 16, fix_body, 0)

    @plsc.parallel_loop(0, NV, unroll=2)
    def _(v):
        sl = _ds16(v)
        g = rarg_b[sl]
        bx1 = plsc.load_gather(bbt, [g])
        by1 = plsc.load_gather(bbt, [g + G])
        bx2 = plsc.load_gather(bbt, [g + 2 * G])
        by2 = plsc.load_gather(bbt, [g + 3 * G])
        ax1, ay1, ax2, ay2 = anc[0, sl], anc[1, sl], anc[2, sl], anc[3, sl]
        w = ax2 - ax1
        h = ay2 - ay1
        ctrx = ax1 + 0.5 * w
        ctry = ay1 + 0.5 * h
        bw = bx2 - bx1
        bh = by2 - by1
        bctrx = bx1 + 0.5 * bw
        bctry = by1 + 0.5 * bh
        wc = jnp.maximum(w, _EPS)
        hc = jnp.maximum(h, _EPS)
        locv[0, sl] = (bctrx - ctrx) / wc
        locv[1, sl] = (bctry - ctry) / hc
        locv[2, sl] = _ln(bw / wc)
        locv[3, sl] = _ln(bh / hc)

    pltpu.sync_copy(locv, loc_hbm.at[:, pl.ds(base, CHUNK)])
    pltpu.sync_copy(lab, label_hbm.at[pl.ds(base, CHUNK)])


@jax.jit
def kernel(anchor, bbox):
    anchor_t = anchor.T
    tail_t = jnp.pad(anchor_t[:, (NW - 1) * CHUNK:],
                     ((0, 0), (0, CHUNK - NLAST)))
    bbox_f = bbox.T.reshape(-1)
    rowmax, rowarg, colval, colidx = _phase1(anchor_t, tail_t, bbox_f)
    loc_t, label = _phase2(anchor_t, tail_t, bbox_f, rowmax, rowarg,
                           colval, colidx)
    return loc_t[:, :N].T, label[:N], rowmax[:N]
